# baseline (speedup 1.0000x reference)
import functools

import jax
import jax.numpy as jnp
from jax import lax
from jax.experimental import pallas as pl
from jax.experimental.pallas import tpu as pltpu
from jax.experimental.pallas import tpu_sc as plsc

N = 10000
E = 160000
D = 256
HALF = 128
NC = 2
NS = 16

RCH = 80
NRCH = N // RCH
RROUNDS = (NRCH + NS - 1) // NS

EPT = E // NS
KS = 50
NCHUNK = EPT // KS
NBUF = 5

DEPT = E // (NC * NS)
KD = 40
NDCHUNK = DEPT // KD

_mesh = plsc.VectorSubcoreMesh(core_axis_name="c", subcore_axis_name="s")


def _row_chunks(s, fn):
    for j in range(RROUNDS):
        k = j * NS + s

        @pl.when(k < NRCH)
        def _():
            fn(k * RCH)


@functools.partial(
    pl.kernel,
    out_type=jax.ShapeDtypeStruct((NC * N, 16), jnp.float32),
    mesh=_mesh,
    scratch_types=[
        pltpu.VMEM((RCH, 16), jnp.float32),
        pltpu.VMEM((NDCHUNK, KD), jnp.int32),
        pltpu.VMEM_SHARED((N, 16), jnp.float32),
        pltpu.SemaphoreType.DMA,
    ],
)
def _sc_degree(dst3_hbm, out_hbm, ones_v, dstall_v, acc, sem):
    c = lax.axis_index("c")
    s = lax.axis_index("s")
    one = jnp.ones((16,), jnp.float32)
    for i in range(RCH):
        ones_v[i, :] = one
    pltpu.sync_copy(dst3_hbm.at[c * NS + s], dstall_v)
    _row_chunks(s, lambda r0: pltpu.sync_copy(
        ones_v, acc.at[pl.ds(r0, RCH)]))
    plsc.subcore_barrier()

    def chunk(i, carry):
        pltpu.sync_copy(ones_v.at[pl.ds(0, KD)], acc.at[dstall_v.at[i]],
                        add=True)
        return carry

    lax.fori_loop(0, NDCHUNK, chunk, 0)
    plsc.subcore_barrier()
    _row_chunks(s, lambda r0: pltpu.sync_copy(
        acc.at[pl.ds(r0, RCH)], out_hbm.at[pl.ds(c * N + r0, RCH)]))


@functools.partial(
    pl.kernel,
    out_type=jax.ShapeDtypeStruct((NC * N, HALF), jnp.float32),
    mesh=_mesh,
    scratch_types=(
        [pltpu.VMEM((KS,), jnp.int32)] * NBUF
        + [pltpu.VMEM((KS,), jnp.int32)] * NBUF
        + [pltpu.VMEM((KS, HALF), jnp.float32)] * NBUF
        + [pltpu.VMEM_SHARED((N, HALF), jnp.float32)]
        + [pltpu.SemaphoreType.DMA] * (4 * NBUF)
    ),
)
def _sc_scatter(y_hbm, src4_hbm, dst3_hbm, out_hbm, *rest):
    srcb = rest[0:NBUF]
    dstb = rest[NBUF:2 * NBUF]
    rowsb = rest[2 * NBUF:3 * NBUF]
    acc = rest[3 * NBUF]
    isem = rest[3 * NBUF + 1:3 * NBUF + 1 + NBUF]
    gsem = rest[3 * NBUF + 1 + NBUF:3 * NBUF + 1 + 2 * NBUF]
    dsem = rest[3 * NBUF + 1 + 2 * NBUF:3 * NBUF + 1 + 3 * NBUF]
    ssem = rest[3 * NBUF + 1 + 3 * NBUF:3 * NBUF + 1 + 4 * NBUF]
    c = lax.axis_index("c")
    s = lax.axis_index("s")
    tid = c * NS + s

    def fetch_idx(j, b):
        pltpu.async_copy(src4_hbm.at[tid, j], srcb[b], isem[b])

    def start_gather(j, b):
        pltpu.make_async_copy(src4_hbm.at[tid, j], srcb[b], isem[b]).wait()
        pltpu.async_copy(y_hbm.at[srcb[b]], rowsb[b], gsem[b])
        pltpu.async_copy(dst3_hbm.at[s, j], dstb[b], dsem[b])

    def wait_gather(j, b):
        pltpu.make_async_copy(y_hbm.at[srcb[b]], rowsb[b], gsem[b]).wait()
        pltpu.make_async_copy(dst3_hbm.at[s, j], dstb[b], dsem[b]).wait()

    def wait_scat(b):
        pltpu.make_async_copy(rowsb[b], acc.at[dstb[b]], ssem[b]).wait()

    for k in range(4):
        fetch_idx(k, k)
    start_gather(0, 0)
    start_gather(1, 1)
    _row_chunks(s, lambda r0: pltpu.sync_copy(
        y_hbm.at[pl.ds(c * N + r0, RCH)], acc.at[pl.ds(r0, RCH)]))
    plsc.subcore_barrier()

    def block(g, carry):
        for b in range(NBUF):
            j = NBUF * g + b
            r = (b + 2) % NBUF
            f = (b + 4) % NBUF
            wait_gather(j, b)
            pltpu.async_copy(rowsb[b], acc.at[dstb[b]], ssem[b], add=True)

            @pl.when(j >= 3)
            def _():
                wait_scat(r)
            start_gather(j + 2, r)

            @pl.when(j + 4 < NCHUNK)
            def _():
                fetch_idx(j + 4, f)
        return carry

    lax.fori_loop(0, NCHUNK // NBUF - 1, block, 0)
    for b in range(NBUF):
        j = NCHUNK - NBUF + b
        wait_gather(j, b)
        pltpu.sync_copy(rowsb[b], acc.at[dstb[b]], add=True)
        if j + 4 < NCHUNK:
            fetch_idx(j + 4, (b + 4) % NBUF)
        if j + 2 < NCHUNK:
            r = (b + 2) % NBUF
            wait_scat(r)
            start_gather(j + 2, r)
    plsc.subcore_barrier()
    _row_chunks(s, lambda r0: pltpu.sync_copy(
        acc.at[pl.ds(r0, RCH)], out_hbm.at[pl.ds(c * N + r0, RCH)]))


BLK = 2000


def _dinv_block(degp_ref):
    d = degp_ref[0, :, 0:1] + degp_ref[1, :, 0:1] - 1.0
    return lax.rsqrt(d)


def _tc_first_body(x_ref, degp_ref, w_ref, y_ref):
    dinv = _dinv_block(degp_ref)
    xw = jnp.dot(x_ref[...], w_ref[...], preferred_element_type=jnp.float32)
    y = xw * dinv
    y_ref[0] = y[:, :HALF]
    y_ref[1] = y[:, HALF:]


def _tc_mid_body(s_ref, degp_ref, b_ref, w_ref, y_ref):
    dinv = _dinv_block(degp_ref)
    h = jnp.concatenate([s_ref[0], s_ref[1]], axis=1) * dinv + b_ref[...]
    y = jnp.dot(h, w_ref[...], preferred_element_type=jnp.float32) * dinv
    y_ref[0] = y[:, :HALF]
    y_ref[1] = y[:, HALF:]


def _tc_final_body(s_ref, degp_ref, b_ref, o_ref):
    dinv = _dinv_block(degp_ref)
    o_ref[...] = jnp.concatenate([s_ref[0], s_ref[1]], axis=1) * dinv + b_ref[...]


_spec_x = pl.BlockSpec((BLK, D), lambda i: (i, 0))
_spec_degp = pl.BlockSpec((2, BLK, 16), lambda i: (0, i, 0))
_spec_w = pl.BlockSpec((D, D), lambda i: (0, 0))
_spec_b = pl.BlockSpec((1, D), lambda i: (0, 0))
_spec_y = pl.BlockSpec((2, BLK, HALF), lambda i: (0, i, 0))

_tc_first = pl.pallas_call(
    _tc_first_body,
    grid=(N // BLK,),
    in_specs=[_spec_x, _spec_degp, _spec_w],
    out_specs=_spec_y,
    out_shape=jax.ShapeDtypeStruct((2, N, HALF), jnp.float32),
)

_tc_mid = pl.pallas_call(
    _tc_mid_body,
    grid=(N // BLK,),
    in_specs=[_spec_y, _spec_degp, _spec_b, _spec_w],
    out_specs=_spec_y,
    out_shape=jax.ShapeDtypeStruct((2, N, HALF), jnp.float32),
)

_tc_final = pl.pallas_call(
    _tc_final_body,
    grid=(N // BLK,),
    in_specs=[_spec_y, _spec_degp, _spec_b],
    out_specs=_spec_x,
    out_shape=jax.ShapeDtypeStruct((N, D), jnp.float32),
)


def kernel(x, edge_index, W1, b1, W2, b2, W3, b3):
    src = edge_index[0].astype(jnp.int32)
    dst = edge_index[1].astype(jnp.int32)
    src4 = jnp.concatenate([src, src + N]).reshape(NC * NS, NCHUNK, KS)
    dst3 = dst.reshape(NS, NCHUNK, KS)
    dstd3 = dst.reshape(NC * NS, NDCHUNK, KD)

    degp = _sc_degree(dstd3).reshape(2, N, 16)

    y = _tc_first(x, degp, W1)
    for b, w in ((b1, W2), (b2, W3)):
        s = _sc_scatter(y.reshape(NC * N, HALF), src4, dst3).reshape(2, N, HALF)
        y = _tc_mid(s, degp, b.reshape(1, D), w)
    s = _sc_scatter(y.reshape(NC * N, HALF), src4, dst3).reshape(2, N, HALF)
    return _tc_final(s, degp, b3.reshape(1, D))


# speedup vs baseline: 1.0978x; 1.0978x over previous
import functools

import jax
import jax.numpy as jnp
from jax import lax
from jax.experimental import pallas as pl
from jax.experimental.pallas import tpu as pltpu
from jax.experimental.pallas import tpu_sc as plsc

N = 10000
E = 160000
D = 256
HALF = 128
NC = 2
NS = 16

RCH = 80
NRCH = N // RCH
RROUNDS = (NRCH + NS - 1) // NS

EPT = E // NS
KS = 80
NCHUNK = EPT // KS
NBUF = 4

DEPT = E // (NC * NS)
KD = 40
NDCHUNK = DEPT // KD

_mesh = plsc.VectorSubcoreMesh(core_axis_name="c", subcore_axis_name="s")


def _row_chunks(s, fn):
    for j in range(RROUNDS):
        k = j * NS + s

        @pl.when(k < NRCH)
        def _():
            fn(k * RCH)


@functools.partial(
    pl.kernel,
    out_type=jax.ShapeDtypeStruct((NC * N, 16), jnp.float32),
    mesh=_mesh,
    scratch_types=[
        pltpu.VMEM((RCH, 16), jnp.float32),
        pltpu.VMEM((NDCHUNK, KD), jnp.int32),
        pltpu.VMEM_SHARED((N, 16), jnp.float32),
        pltpu.SemaphoreType.DMA,
    ],
)
def _sc_degree(dst3_hbm, out_hbm, ones_v, dstall_v, acc, sem):
    c = lax.axis_index("c")
    s = lax.axis_index("s")
    one = jnp.ones((16,), jnp.float32)
    for i in range(RCH):
        ones_v[i, :] = one
    pltpu.sync_copy(dst3_hbm.at[c * NS + s], dstall_v)
    _row_chunks(s, lambda r0: pltpu.sync_copy(
        ones_v, acc.at[pl.ds(r0, RCH)]))
    plsc.subcore_barrier()

    def chunk(i, carry):
        pltpu.sync_copy(ones_v.at[pl.ds(0, KD)], acc.at[dstall_v.at[i]],
                        add=True)
        return carry

    lax.fori_loop(0, NDCHUNK, chunk, 0)
    plsc.subcore_barrier()
    _row_chunks(s, lambda r0: pltpu.sync_copy(
        acc.at[pl.ds(r0, RCH)], out_hbm.at[pl.ds(c * N + r0, RCH)]))


@functools.partial(
    pl.kernel,
    out_type=jax.ShapeDtypeStruct((NC * N, HALF), jnp.float32),
    mesh=_mesh,
    scratch_types=(
        [pltpu.VMEM((1, KS), jnp.int32)] * NBUF
        + [pltpu.VMEM((1, KS), jnp.int32)] * NBUF
        + [pltpu.VMEM((KS, HALF), jnp.float32)] * NBUF
        + [pltpu.VMEM_SHARED((N, HALF), jnp.float32)]
        + [pltpu.SemaphoreType.DMA] * (4 * NBUF)
    ),
)
def _sc_scatter(y_hbm, src4_hbm, dst3_hbm, out_hbm, *rest):
    srcb = rest[0:NBUF]
    dstb = rest[NBUF:2 * NBUF]
    rowsb = rest[2 * NBUF:3 * NBUF]
    acc = rest[3 * NBUF]
    isem = rest[3 * NBUF + 1:3 * NBUF + 1 + NBUF]
    gsem = rest[3 * NBUF + 1 + NBUF:3 * NBUF + 1 + 2 * NBUF]
    dsem = rest[3 * NBUF + 1 + 2 * NBUF:3 * NBUF + 1 + 3 * NBUF]
    ssem = rest[3 * NBUF + 1 + 3 * NBUF:3 * NBUF + 1 + 4 * NBUF]
    c = lax.axis_index("c")
    s = lax.axis_index("s")
    tid = c * NS + s

    def fetch_idx(j, b):
        pltpu.async_copy(src4_hbm.at[tid * NCHUNK + j], srcb[b], isem[b])

    def start_gather(j, b):
        pltpu.make_async_copy(src4_hbm.at[tid * NCHUNK + j], srcb[b],
                              isem[b]).wait()
        pltpu.async_copy(y_hbm.at[srcb[b].at[0]], rowsb[b], gsem[b])
        pltpu.async_copy(dst3_hbm.at[s * NCHUNK + j], dstb[b], dsem[b])

    def wait_gather(j, b):
        pltpu.make_async_copy(y_hbm.at[srcb[b].at[0]], rowsb[b], gsem[b]).wait()
        pltpu.make_async_copy(dst3_hbm.at[s * NCHUNK + j], dstb[b],
                              dsem[b]).wait()

    def wait_scat(b):
        pltpu.make_async_copy(rowsb[b], acc.at[dstb[b].at[0]], ssem[b]).wait()

    for k in range(NBUF):
        fetch_idx(k, k)
    start_gather(0, 0)
    start_gather(1, 1)
    _row_chunks(s, lambda r0: pltpu.sync_copy(
        y_hbm.at[pl.ds(c * N + r0, RCH)], acc.at[pl.ds(r0, RCH)]))
    plsc.subcore_barrier()

    NMAIN = (NCHUNK // NBUF - 1) * NBUF

    def block(g, carry):
        for b in range(NBUF):
            j = NBUF * g + b
            r = (b + 2) % NBUF
            wait_gather(j, b)
            fetch_idx(j + NBUF, b)
            pltpu.async_copy(rowsb[b], acc.at[dstb[b].at[0]], ssem[b], add=True)

            @pl.when(j >= 2)
            def _():
                wait_scat(r)
            start_gather(j + 2, r)
        return carry

    lax.fori_loop(0, NMAIN // NBUF, block, 0)
    for j in range(NMAIN, NCHUNK):
        b = j % NBUF
        wait_gather(j, b)
        if j + NBUF < NCHUNK:
            fetch_idx(j + NBUF, b)
        pltpu.sync_copy(rowsb[b], acc.at[dstb[b].at[0]], add=True)
        if j + 2 < NCHUNK:
            r = (j + 2) % NBUF
            if j - 2 < NMAIN:
                wait_scat(r)
            start_gather(j + 2, r)
    plsc.subcore_barrier()
    _row_chunks(s, lambda r0: pltpu.sync_copy(
        acc.at[pl.ds(r0, RCH)], out_hbm.at[pl.ds(c * N + r0, RCH)]))


BLK = 2000


def _dinv_block(degp_ref):
    d = degp_ref[0, :, 0:1] + degp_ref[1, :, 0:1] - 1.0
    return lax.rsqrt(d)


def _tc_first_body(x_ref, degp_ref, w_ref, y_ref):
    dinv = _dinv_block(degp_ref)
    xw = jnp.dot(x_ref[...], w_ref[...], preferred_element_type=jnp.float32)
    y = xw * dinv
    y_ref[0] = y[:, :HALF]
    y_ref[1] = y[:, HALF:]


def _tc_mid_body(s_ref, degp_ref, b_ref, w_ref, y_ref):
    dinv = _dinv_block(degp_ref)
    h = jnp.concatenate([s_ref[0], s_ref[1]], axis=1) * dinv + b_ref[...]
    y = jnp.dot(h, w_ref[...], preferred_element_type=jnp.float32) * dinv
    y_ref[0] = y[:, :HALF]
    y_ref[1] = y[:, HALF:]


def _tc_final_body(s_ref, degp_ref, b_ref, o_ref):
    dinv = _dinv_block(degp_ref)
    o_ref[...] = jnp.concatenate([s_ref[0], s_ref[1]], axis=1) * dinv + b_ref[...]


_spec_x = pl.BlockSpec((BLK, D), lambda i: (i, 0))
_spec_degp = pl.BlockSpec((2, BLK, 16), lambda i: (0, i, 0))
_spec_w = pl.BlockSpec((D, D), lambda i: (0, 0))
_spec_b = pl.BlockSpec((1, D), lambda i: (0, 0))
_spec_y = pl.BlockSpec((2, BLK, HALF), lambda i: (0, i, 0))

_tc_first = pl.pallas_call(
    _tc_first_body,
    grid=(N // BLK,),
    in_specs=[_spec_x, _spec_degp, _spec_w],
    out_specs=_spec_y,
    out_shape=jax.ShapeDtypeStruct((2, N, HALF), jnp.float32),
)

_tc_mid = pl.pallas_call(
    _tc_mid_body,
    grid=(N // BLK,),
    in_specs=[_spec_y, _spec_degp, _spec_b, _spec_w],
    out_specs=_spec_y,
    out_shape=jax.ShapeDtypeStruct((2, N, HALF), jnp.float32),
)

_tc_final = pl.pallas_call(
    _tc_final_body,
    grid=(N // BLK,),
    in_specs=[_spec_y, _spec_degp, _spec_b],
    out_specs=_spec_x,
    out_shape=jax.ShapeDtypeStruct((N, D), jnp.float32),
)


def kernel(x, edge_index, W1, b1, W2, b2, W3, b3):
    src = edge_index[0].astype(jnp.int32)
    dst = edge_index[1].astype(jnp.int32)
    src4 = jnp.concatenate([src, src + N]).reshape(NC * NS * NCHUNK, 1, KS)
    dst3 = dst.reshape(NS * NCHUNK, 1, KS)
    dstd3 = dst.reshape(NC * NS, NDCHUNK, KD)

    degp = _sc_degree(dstd3).reshape(2, N, 16)

    y = _tc_first(x, degp, W1)
    for b, w in ((b1, W2), (b2, W3)):
        s = _sc_scatter(y.reshape(NC * N, HALF), src4, dst3).reshape(2, N, HALF)
        y = _tc_mid(s, degp, b.reshape(1, D), w)
    s = _sc_scatter(y.reshape(NC * N, HALF), src4, dst3).reshape(2, N, HALF)
    return _tc_final(s, degp, b3.reshape(1, D))


# speedup vs baseline: 1.1701x; 1.0659x over previous
import functools

import jax
import jax.numpy as jnp
from jax import lax
from jax.experimental import pallas as pl
from jax.experimental.pallas import tpu as pltpu
from jax.experimental.pallas import tpu_sc as plsc

N = 10000
E = 160000
D = 256
HALF = 128
NC = 2
NS = 16

RCH = 80
NRCH = N // RCH
RROUNDS = (NRCH + NS - 1) // NS

EPT = E // NS
KS = 125
NCHUNK = EPT // KS

DEPT = E // (NC * NS)
KD = 40
NDCHUNK = DEPT // KD

_mesh = plsc.VectorSubcoreMesh(core_axis_name="c", subcore_axis_name="s")


def _row_chunks(s, fn):
    for j in range(RROUNDS):
        k = j * NS + s

        @pl.when(k < NRCH)
        def _():
            fn(k * RCH)


@functools.partial(
    pl.kernel,
    out_type=jax.ShapeDtypeStruct((NC * N, 16), jnp.float32),
    mesh=_mesh,
    scratch_types=[
        pltpu.VMEM((RCH, 16), jnp.float32),
        pltpu.VMEM((NDCHUNK, KD), jnp.int32),
        pltpu.VMEM_SHARED((N, 16), jnp.float32),
        pltpu.SemaphoreType.DMA,
    ],
)
def _sc_degree(dst3_hbm, out_hbm, ones_v, dstall_v, acc, sem):
    c = lax.axis_index("c")
    s = lax.axis_index("s")
    one = jnp.ones((16,), jnp.float32)
    for i in range(RCH):
        ones_v[i, :] = one
    pltpu.sync_copy(dst3_hbm.at[c * NS + s], dstall_v)
    _row_chunks(s, lambda r0: pltpu.sync_copy(
        ones_v, acc.at[pl.ds(r0, RCH)]))
    plsc.subcore_barrier()

    def chunk(i, carry):
        pltpu.sync_copy(ones_v.at[pl.ds(0, KD)], acc.at[dstall_v.at[i]],
                        add=True)
        return carry

    lax.fori_loop(0, NDCHUNK, chunk, 0)
    plsc.subcore_barrier()
    _row_chunks(s, lambda r0: pltpu.sync_copy(
        acc.at[pl.ds(r0, RCH)], out_hbm.at[pl.ds(c * N + r0, RCH)]))


@functools.partial(
    pl.kernel,
    out_type=jax.ShapeDtypeStruct((NC * N, HALF), jnp.float32),
    mesh=_mesh,
    scratch_types=(
        [pltpu.VMEM((NCHUNK, KS), jnp.int32)]
        + [pltpu.VMEM((1, KS), jnp.int32)] * 2
        + [pltpu.VMEM((KS, HALF), jnp.float32)] * 2
        + [pltpu.VMEM_SHARED((N, HALF), jnp.float32)]
        + [pltpu.SemaphoreType.DMA] * 4
    ),
)
def _sc_scatter(y_hbm, src3_hbm, dst3_hbm, out_hbm, srcall_v,
                dst0_v, dst1_v, rows0_v, rows1_v, acc,
                gsem0, gsem1, dsem0, dsem1):
    c = lax.axis_index("c")
    s = lax.axis_index("s")
    tid = c * NS + s
    pltpu.sync_copy(src3_hbm.at[tid], srcall_v)

    def fetch(j, dst_v, rows_v, gsem, dsem):
        pltpu.async_copy(y_hbm.at[srcall_v.at[j]], rows_v, gsem)
        pltpu.async_copy(dst3_hbm.at[s * NCHUNK + j], dst_v, dsem)

    def wait_fetch(j, dst_v, rows_v, gsem, dsem):
        pltpu.make_async_copy(y_hbm.at[srcall_v.at[j]], rows_v, gsem).wait()
        pltpu.make_async_copy(dst3_hbm.at[s * NCHUNK + j], dst_v, dsem).wait()

    fetch(0, dst0_v, rows0_v, gsem0, dsem0)
    fetch(1, dst1_v, rows1_v, gsem1, dsem1)
    _row_chunks(s, lambda r0: pltpu.sync_copy(
        y_hbm.at[pl.ds(c * N + r0, RCH)], acc.at[pl.ds(r0, RCH)]))
    plsc.subcore_barrier()

    bufs = ((dst0_v, rows0_v, gsem0, dsem0), (dst1_v, rows1_v, gsem1, dsem1))

    def pair(g, carry):
        for b, (dst_v, rows_v, gsem, dsem) in enumerate(bufs):
            j = 2 * g + b
            wait_fetch(j, dst_v, rows_v, gsem, dsem)
            pltpu.sync_copy(rows_v, acc.at[dst_v.at[0]], add=True)

            @pl.when(j + 2 < NCHUNK)
            def _():
                fetch(j + 2, dst_v, rows_v, gsem, dsem)
        return carry

    lax.fori_loop(0, NCHUNK // 2, pair, 0)
    plsc.subcore_barrier()
    _row_chunks(s, lambda r0: pltpu.sync_copy(
        acc.at[pl.ds(r0, RCH)], out_hbm.at[pl.ds(c * N + r0, RCH)]))


BLK = 2000


def _dinv_block(degp_ref):
    d = degp_ref[0, :, 0:1] + degp_ref[1, :, 0:1] - 1.0
    return lax.rsqrt(d)


def _tc_first_body(x_ref, degp_ref, w_ref, y_ref):
    dinv = _dinv_block(degp_ref)
    xw = jnp.dot(x_ref[...], w_ref[...], preferred_element_type=jnp.float32)
    y = xw * dinv
    y_ref[0] = y[:, :HALF]
    y_ref[1] = y[:, HALF:]


def _tc_mid_body(s_ref, degp_ref, b_ref, w_ref, y_ref):
    dinv = _dinv_block(degp_ref)
    h = jnp.concatenate([s_ref[0], s_ref[1]], axis=1) * dinv + b_ref[...]
    y = jnp.dot(h, w_ref[...], preferred_element_type=jnp.float32) * dinv
    y_ref[0] = y[:, :HALF]
    y_ref[1] = y[:, HALF:]


def _tc_final_body(s_ref, degp_ref, b_ref, o_ref):
    dinv = _dinv_block(degp_ref)
    o_ref[...] = jnp.concatenate([s_ref[0], s_ref[1]], axis=1) * dinv + b_ref[...]


_spec_x = pl.BlockSpec((BLK, D), lambda i: (i, 0))
_spec_degp = pl.BlockSpec((2, BLK, 16), lambda i: (0, i, 0))
_spec_w = pl.BlockSpec((D, D), lambda i: (0, 0))
_spec_b = pl.BlockSpec((1, D), lambda i: (0, 0))
_spec_y = pl.BlockSpec((2, BLK, HALF), lambda i: (0, i, 0))

_tc_first = pl.pallas_call(
    _tc_first_body,
    grid=(N // BLK,),
    in_specs=[_spec_x, _spec_degp, _spec_w],
    out_specs=_spec_y,
    out_shape=jax.ShapeDtypeStruct((2, N, HALF), jnp.float32),
)

_tc_mid = pl.pallas_call(
    _tc_mid_body,
    grid=(N // BLK,),
    in_specs=[_spec_y, _spec_degp, _spec_b, _spec_w],
    out_specs=_spec_y,
    out_shape=jax.ShapeDtypeStruct((2, N, HALF), jnp.float32),
)

_tc_final = pl.pallas_call(
    _tc_final_body,
    grid=(N // BLK,),
    in_specs=[_spec_y, _spec_degp, _spec_b],
    out_specs=_spec_x,
    out_shape=jax.ShapeDtypeStruct((N, D), jnp.float32),
)


def kernel(x, edge_index, W1, b1, W2, b2, W3, b3):
    src = edge_index[0].astype(jnp.int32)
    dst = edge_index[1].astype(jnp.int32)
    src3 = jnp.concatenate([src, src + N]).reshape(NC * NS, NCHUNK, KS)
    dst3 = dst.reshape(NS * NCHUNK, 1, KS)
    dstd3 = dst.reshape(NC * NS, NDCHUNK, KD)

    degp = _sc_degree(dstd3).reshape(2, N, 16)

    y = _tc_first(x, degp, W1)
    for b, w in ((b1, W2), (b2, W3)):
        s = _sc_scatter(y.reshape(NC * N, HALF), src3, dst3).reshape(2, N, HALF)
        y = _tc_mid(s, degp, b.reshape(1, D), w)
    s = _sc_scatter(y.reshape(NC * N, HALF), src3, dst3).reshape(2, N, HALF)
    return _tc_final(s, degp, b3.reshape(1, D))


# speedup vs baseline: 1.1853x; 1.0129x over previous
import functools

import jax
import jax.numpy as jnp
from jax import lax
from jax.experimental import pallas as pl
from jax.experimental.pallas import tpu as pltpu
from jax.experimental.pallas import tpu_sc as plsc

N = 10000
E = 160000
D = 256
HALF = 128
NC = 2
NS = 16

RCH = 80
NRCH = N // RCH
RROUNDS = (NRCH + NS - 1) // NS

EPT = E // NS
KS = 125
NCHUNK = EPT // KS

DEPT = E // (NC * NS)
KD = 40
NDCHUNK = DEPT // KD

_mesh = plsc.VectorSubcoreMesh(core_axis_name="c", subcore_axis_name="s")


def _row_chunks(s, fn):
    for j in range(RROUNDS):
        k = j * NS + s

        @pl.when(k < NRCH)
        def _():
            fn(k * RCH)


@functools.partial(
    pl.kernel,
    out_type=jax.ShapeDtypeStruct((NC * N, 16), jnp.float32),
    mesh=_mesh,
    scratch_types=[
        pltpu.VMEM((RCH, 16), jnp.float32),
        pltpu.VMEM((NDCHUNK, KD), jnp.int32),
        pltpu.VMEM_SHARED((N, 16), jnp.float32),
        pltpu.SemaphoreType.DMA,
    ],
)
def _sc_degree(dst3_hbm, out_hbm, ones_v, dstall_v, acc, sem):
    c = lax.axis_index("c")
    s = lax.axis_index("s")
    one = jnp.ones((16,), jnp.float32)
    for i in range(RCH):
        ones_v[i, :] = one
    pltpu.sync_copy(dst3_hbm.at[c * NS + s], dstall_v)
    _row_chunks(s, lambda r0: pltpu.sync_copy(
        ones_v, acc.at[pl.ds(r0, RCH)]))
    plsc.subcore_barrier()

    def chunk(i, carry):
        @pl.when(i >= 2)
        def _():
            pltpu.make_async_copy(ones_v.at[pl.ds(0, KD)],
                                  acc.at[dstall_v.at[0]], sem).wait()
        pltpu.async_copy(ones_v.at[pl.ds(0, KD)], acc.at[dstall_v.at[i]], sem,
                        add=True)
        return carry

    lax.fori_loop(0, NDCHUNK, chunk, 0)
    for _ in range(2):
        pltpu.make_async_copy(ones_v.at[pl.ds(0, KD)],
                              acc.at[dstall_v.at[0]], sem).wait()
    plsc.subcore_barrier()
    _row_chunks(s, lambda r0: pltpu.sync_copy(
        acc.at[pl.ds(r0, RCH)], out_hbm.at[pl.ds(c * N + r0, RCH)]))


@functools.partial(
    pl.kernel,
    out_type=jax.ShapeDtypeStruct((NC * N, HALF), jnp.float32),
    mesh=_mesh,
    scratch_types=(
        [pltpu.VMEM((NCHUNK, KS), jnp.int32)]
        + [pltpu.VMEM((1, KS), jnp.int32)] * 2
        + [pltpu.VMEM((KS, HALF), jnp.float32)] * 2
        + [pltpu.VMEM_SHARED((N, HALF), jnp.float32)]
        + [pltpu.SemaphoreType.DMA] * 4
    ),
)
def _sc_scatter(y_hbm, src3_hbm, dst3_hbm, out_hbm, srcall_v,
                dst0_v, dst1_v, rows0_v, rows1_v, acc,
                gsem0, gsem1, dsem0, dsem1):
    c = lax.axis_index("c")
    s = lax.axis_index("s")
    tid = c * NS + s
    pltpu.sync_copy(src3_hbm.at[tid], srcall_v)

    def fetch(j, dst_v, rows_v, gsem, dsem):
        pltpu.async_copy(y_hbm.at[srcall_v.at[j]], rows_v, gsem)
        pltpu.async_copy(dst3_hbm.at[s * NCHUNK + j], dst_v, dsem)

    def wait_fetch(j, dst_v, rows_v, gsem, dsem):
        pltpu.make_async_copy(y_hbm.at[srcall_v.at[j]], rows_v, gsem).wait()
        pltpu.make_async_copy(dst3_hbm.at[s * NCHUNK + j], dst_v, dsem).wait()

    fetch(0, dst0_v, rows0_v, gsem0, dsem0)
    fetch(1, dst1_v, rows1_v, gsem1, dsem1)
    _row_chunks(s, lambda r0: pltpu.sync_copy(
        y_hbm.at[pl.ds(c * N + r0, RCH)], acc.at[pl.ds(r0, RCH)]))
    plsc.subcore_barrier()

    bufs = ((dst0_v, rows0_v, gsem0, dsem0), (dst1_v, rows1_v, gsem1, dsem1))

    def pair(g, carry):
        for b, (dst_v, rows_v, gsem, dsem) in enumerate(bufs):
            j = 2 * g + b
            wait_fetch(j, dst_v, rows_v, gsem, dsem)
            pltpu.sync_copy(rows_v, acc.at[dst_v.at[0]], add=True)

            @pl.when(j + 2 < NCHUNK)
            def _():
                fetch(j + 2, dst_v, rows_v, gsem, dsem)
        return carry

    lax.fori_loop(0, NCHUNK // 2, pair, 0)
    plsc.subcore_barrier()
    _row_chunks(s, lambda r0: pltpu.sync_copy(
        acc.at[pl.ds(r0, RCH)], out_hbm.at[pl.ds(c * N + r0, RCH)]))


BLK = 2000


def _dinv_block(degp_ref):
    d = degp_ref[0, :, 0:1] + degp_ref[1, :, 0:1] - 1.0
    return lax.rsqrt(d)


def _tc_first_body(x_ref, degp_ref, w_ref, y_ref):
    dinv = _dinv_block(degp_ref)
    xw = jnp.dot(x_ref[...], w_ref[...], preferred_element_type=jnp.float32)
    y = xw * dinv
    y_ref[0] = y[:, :HALF]
    y_ref[1] = y[:, HALF:]


def _tc_mid_body(s_ref, degp_ref, b_ref, w_ref, y_ref):
    dinv = _dinv_block(degp_ref)
    h = jnp.concatenate([s_ref[0], s_ref[1]], axis=1) * dinv + b_ref[...]
    y = jnp.dot(h, w_ref[...], preferred_element_type=jnp.float32) * dinv
    y_ref[0] = y[:, :HALF]
    y_ref[1] = y[:, HALF:]


def _tc_final_body(s_ref, degp_ref, b_ref, o_ref):
    dinv = _dinv_block(degp_ref)
    o_ref[...] = jnp.concatenate([s_ref[0], s_ref[1]], axis=1) * dinv + b_ref[...]


_spec_x = pl.BlockSpec((BLK, D), lambda i: (i, 0))
_spec_degp = pl.BlockSpec((2, BLK, 16), lambda i: (0, i, 0))
_spec_w = pl.BlockSpec((D, D), lambda i: (0, 0))
_spec_b = pl.BlockSpec((1, D), lambda i: (0, 0))
_spec_y = pl.BlockSpec((2, BLK, HALF), lambda i: (0, i, 0))

_tc_first = pl.pallas_call(
    _tc_first_body,
    grid=(N // BLK,),
    in_specs=[_spec_x, _spec_degp, _spec_w],
    out_specs=_spec_y,
    out_shape=jax.ShapeDtypeStruct((2, N, HALF), jnp.float32),
)

_tc_mid = pl.pallas_call(
    _tc_mid_body,
    grid=(N // BLK,),
    in_specs=[_spec_y, _spec_degp, _spec_b, _spec_w],
    out_specs=_spec_y,
    out_shape=jax.ShapeDtypeStruct((2, N, HALF), jnp.float32),
)

_tc_final = pl.pallas_call(
    _tc_final_body,
    grid=(N // BLK,),
    in_specs=[_spec_y, _spec_degp, _spec_b],
    out_specs=_spec_x,
    out_shape=jax.ShapeDtypeStruct((N, D), jnp.float32),
)


def kernel(x, edge_index, W1, b1, W2, b2, W3, b3):
    src = edge_index[0].astype(jnp.int32)
    dst = edge_index[1].astype(jnp.int32)
    src3 = jnp.concatenate([src, src + N]).reshape(NC * NS, NCHUNK, KS)
    dst3 = dst.reshape(NS * NCHUNK, 1, KS)
    dstd3 = dst.reshape(NC * NS, NDCHUNK, KD)

    degp = _sc_degree(dstd3).reshape(2, N, 16)

    y = _tc_first(x, degp, W1)
    for b, w in ((b1, W2), (b2, W3)):
        s = _sc_scatter(y.reshape(NC * N, HALF), src3, dst3).reshape(2, N, HALF)
        y = _tc_mid(s, degp, b.reshape(1, D), w)
    s = _sc_scatter(y.reshape(NC * N, HALF), src3, dst3).reshape(2, N, HALF)
    return _tc_final(s, degp, b3.reshape(1, D))


# speedup vs baseline: 1.1915x; 1.0052x over previous
import functools

import jax
import jax.numpy as jnp
from jax import lax
from jax.experimental import pallas as pl
from jax.experimental.pallas import tpu as pltpu
from jax.experimental.pallas import tpu_sc as plsc

N = 10000
E = 160000
D = 256
HALF = 128
NC = 2
NS = 16

RCH = 80
NRCH = N // RCH
RROUNDS = (NRCH + NS - 1) // NS

EPT = E // NS
KS = 125
NCHUNK = EPT // KS

DEPT = E // (NC * NS)
KD = 40
NDCHUNK = DEPT // KD

_mesh = plsc.VectorSubcoreMesh(core_axis_name="c", subcore_axis_name="s")


def _row_chunks(s, fn):
    for j in range(RROUNDS):
        k = j * NS + s

        @pl.when(k < NRCH)
        def _():
            fn(k * RCH)


@functools.partial(
    pl.kernel,
    out_type=jax.ShapeDtypeStruct((NC * N, 16), jnp.float32),
    mesh=_mesh,
    scratch_types=[
        pltpu.VMEM((RCH, 16), jnp.float32),
        pltpu.VMEM((NDCHUNK, KD), jnp.int32),
        pltpu.VMEM_SHARED((N, 16), jnp.float32),
        pltpu.SemaphoreType.DMA,
    ],
)
def _sc_degree(dst3_hbm, out_hbm, ones_v, dstall_v, acc, sem):
    c = lax.axis_index("c")
    s = lax.axis_index("s")
    one = jnp.ones((16,), jnp.float32)
    for i in range(RCH):
        ones_v[i, :] = one
    pltpu.sync_copy(dst3_hbm.at[c * NS + s], dstall_v)
    _row_chunks(s, lambda r0: pltpu.sync_copy(
        ones_v, acc.at[pl.ds(r0, RCH)]))
    plsc.subcore_barrier()

    def chunk(i, carry):
        @pl.when(i >= 4)
        def _():
            pltpu.make_async_copy(ones_v.at[pl.ds(0, KD)],
                                  acc.at[dstall_v.at[0]], sem).wait()
        pltpu.async_copy(ones_v.at[pl.ds(0, KD)], acc.at[dstall_v.at[i]], sem,
                        add=True)
        return carry

    lax.fori_loop(0, NDCHUNK, chunk, 0)
    for _ in range(4):
        pltpu.make_async_copy(ones_v.at[pl.ds(0, KD)],
                              acc.at[dstall_v.at[0]], sem).wait()
    plsc.subcore_barrier()
    _row_chunks(s, lambda r0: pltpu.sync_copy(
        acc.at[pl.ds(r0, RCH)], out_hbm.at[pl.ds(c * N + r0, RCH)]))


@functools.partial(
    pl.kernel,
    out_type=jax.ShapeDtypeStruct((NC * N, HALF), jnp.float32),
    mesh=_mesh,
    scratch_types=(
        [pltpu.VMEM((NCHUNK, KS), jnp.int32)]
        + [pltpu.VMEM((1, KS), jnp.int32)] * 2
        + [pltpu.VMEM((KS, HALF), jnp.float32)] * 2
        + [pltpu.VMEM_SHARED((N, HALF), jnp.float32)]
        + [pltpu.SemaphoreType.DMA] * 4
    ),
)
def _sc_scatter(y_hbm, src3_hbm, dst3_hbm, out_hbm, srcall_v,
                dst0_v, dst1_v, rows0_v, rows1_v, acc,
                gsem0, gsem1, dsem0, dsem1):
    c = lax.axis_index("c")
    s = lax.axis_index("s")
    tid = c * NS + s
    pltpu.sync_copy(src3_hbm.at[tid], srcall_v)

    def fetch(j, dst_v, rows_v, gsem, dsem):
        pltpu.async_copy(y_hbm.at[srcall_v.at[j]], rows_v, gsem)
        pltpu.async_copy(dst3_hbm.at[s * NCHUNK + j], dst_v, dsem)

    def wait_fetch(j, dst_v, rows_v, gsem, dsem):
        pltpu.make_async_copy(y_hbm.at[srcall_v.at[j]], rows_v, gsem).wait()
        pltpu.make_async_copy(dst3_hbm.at[s * NCHUNK + j], dst_v, dsem).wait()

    fetch(0, dst0_v, rows0_v, gsem0, dsem0)
    fetch(1, dst1_v, rows1_v, gsem1, dsem1)
    _row_chunks(s, lambda r0: pltpu.sync_copy(
        y_hbm.at[pl.ds(c * N + r0, RCH)], acc.at[pl.ds(r0, RCH)]))
    plsc.subcore_barrier()

    bufs = ((dst0_v, rows0_v, gsem0, dsem0), (dst1_v, rows1_v, gsem1, dsem1))

    def pair(g, carry):
        for b, (dst_v, rows_v, gsem, dsem) in enumerate(bufs):
            j = 2 * g + b
            wait_fetch(j, dst_v, rows_v, gsem, dsem)
            pltpu.sync_copy(rows_v, acc.at[dst_v.at[0]], add=True)

            @pl.when(j + 2 < NCHUNK)
            def _():
                fetch(j + 2, dst_v, rows_v, gsem, dsem)
        return carry

    lax.fori_loop(0, NCHUNK // 2, pair, 0)
    plsc.subcore_barrier()
    _row_chunks(s, lambda r0: pltpu.sync_copy(
        acc.at[pl.ds(r0, RCH)], out_hbm.at[pl.ds(c * N + r0, RCH)]))


BLK = 2000


def _dinv_block(degp_ref):
    d = degp_ref[0, :, 0:1] + degp_ref[1, :, 0:1] - 1.0
    return lax.rsqrt(d)


def _tc_xw_body(x_ref, w_ref, o_ref):
    o_ref[...] = jnp.dot(x_ref[...], w_ref[...],
                         preferred_element_type=jnp.float32)


def _tc_scale_body(xw_ref, degp_ref, y_ref):
    dinv = _dinv_block(degp_ref)
    y = xw_ref[...] * dinv
    y_ref[0] = y[:, :HALF]
    y_ref[1] = y[:, HALF:]


def _tc_mid_body(s_ref, degp_ref, b_ref, w_ref, y_ref):
    dinv = _dinv_block(degp_ref)
    h = jnp.concatenate([s_ref[0], s_ref[1]], axis=1) * dinv + b_ref[...]
    y = jnp.dot(h, w_ref[...], preferred_element_type=jnp.float32) * dinv
    y_ref[0] = y[:, :HALF]
    y_ref[1] = y[:, HALF:]


def _tc_final_body(s_ref, degp_ref, b_ref, o_ref):
    dinv = _dinv_block(degp_ref)
    o_ref[...] = jnp.concatenate([s_ref[0], s_ref[1]], axis=1) * dinv + b_ref[...]


_spec_x = pl.BlockSpec((BLK, D), lambda i: (i, 0))
_spec_degp = pl.BlockSpec((2, BLK, 16), lambda i: (0, i, 0))
_spec_w = pl.BlockSpec((D, D), lambda i: (0, 0))
_spec_b = pl.BlockSpec((1, D), lambda i: (0, 0))
_spec_y = pl.BlockSpec((2, BLK, HALF), lambda i: (0, i, 0))

_tc_xw = pl.pallas_call(
    _tc_xw_body,
    grid=(N // BLK,),
    in_specs=[_spec_x, _spec_w],
    out_specs=_spec_x,
    out_shape=jax.ShapeDtypeStruct((N, D), jnp.float32),
)

_tc_scale = pl.pallas_call(
    _tc_scale_body,
    grid=(N // BLK,),
    in_specs=[_spec_x, _spec_degp],
    out_specs=_spec_y,
    out_shape=jax.ShapeDtypeStruct((2, N, HALF), jnp.float32),
)

_tc_mid = pl.pallas_call(
    _tc_mid_body,
    grid=(N // BLK,),
    in_specs=[_spec_y, _spec_degp, _spec_b, _spec_w],
    out_specs=_spec_y,
    out_shape=jax.ShapeDtypeStruct((2, N, HALF), jnp.float32),
)

_tc_final = pl.pallas_call(
    _tc_final_body,
    grid=(N // BLK,),
    in_specs=[_spec_y, _spec_degp, _spec_b],
    out_specs=_spec_x,
    out_shape=jax.ShapeDtypeStruct((N, D), jnp.float32),
)


def kernel(x, edge_index, W1, b1, W2, b2, W3, b3):
    src = edge_index[0].astype(jnp.int32)
    dst = edge_index[1].astype(jnp.int32)
    src3 = jnp.concatenate([src, src + N]).reshape(NC * NS, NCHUNK, KS)
    dst3 = dst.reshape(NS * NCHUNK, 1, KS)
    dstd3 = dst.reshape(NC * NS, NDCHUNK, KD)

    xw1 = _tc_xw(x, W1)
    degp = _sc_degree(dstd3).reshape(2, N, 16)
    y = _tc_scale(xw1, degp)
    for b, w in ((b1, W2), (b2, W3)):
        s = _sc_scatter(y.reshape(NC * N, HALF), src3, dst3).reshape(2, N, HALF)
        y = _tc_mid(s, degp, b.reshape(1, D), w)
    s = _sc_scatter(y.reshape(NC * N, HALF), src3, dst3).reshape(2, N, HALF)
    return _tc_final(s, degp, b3.reshape(1, D))


# speedup vs baseline: 1.2129x; 1.0180x over previous
import functools

import jax
import jax.numpy as jnp
from jax import lax
from jax.experimental import pallas as pl
from jax.experimental.pallas import tpu as pltpu
from jax.experimental.pallas import tpu_sc as plsc

N = 10000
E = 160000
D = 256
HALF = 128
NC = 2
NS = 16

RCH = 80
NRCH = N // RCH
RROUNDS = (NRCH + NS - 1) // NS

EPT = E // NS
KS = 100
NCHUNK = EPT // KS
NBUF = 3

DEPT = E // (NC * NS)
KD = 40
NDCHUNK = DEPT // KD

_mesh = plsc.VectorSubcoreMesh(core_axis_name="c", subcore_axis_name="s")


def _row_chunks(s, fn):
    for j in range(RROUNDS):
        k = j * NS + s

        @pl.when(k < NRCH)
        def _():
            fn(k * RCH)


@functools.partial(
    pl.kernel,
    out_type=jax.ShapeDtypeStruct((NC * N, 16), jnp.float32),
    mesh=_mesh,
    scratch_types=[
        pltpu.VMEM((RCH, 16), jnp.float32),
        pltpu.VMEM((NDCHUNK, KD), jnp.int32),
        pltpu.VMEM_SHARED((N, 16), jnp.float32),
        pltpu.SemaphoreType.DMA,
    ],
)
def _sc_degree(dst3_hbm, out_hbm, ones_v, dstall_v, acc, sem):
    c = lax.axis_index("c")
    s = lax.axis_index("s")
    one = jnp.ones((16,), jnp.float32)
    for i in range(RCH):
        ones_v[i, :] = one
    pltpu.sync_copy(dst3_hbm.at[c * NS + s], dstall_v)
    _row_chunks(s, lambda r0: pltpu.sync_copy(
        ones_v, acc.at[pl.ds(r0, RCH)]))
    plsc.subcore_barrier()

    def chunk(i, carry):
        @pl.when(i >= 4)
        def _():
            pltpu.make_async_copy(ones_v.at[pl.ds(0, KD)],
                                  acc.at[dstall_v.at[0]], sem).wait()
        pltpu.async_copy(ones_v.at[pl.ds(0, KD)], acc.at[dstall_v.at[i]], sem,
                        add=True)
        return carry

    lax.fori_loop(0, NDCHUNK, chunk, 0)
    for _ in range(4):
        pltpu.make_async_copy(ones_v.at[pl.ds(0, KD)],
                              acc.at[dstall_v.at[0]], sem).wait()
    plsc.subcore_barrier()
    _row_chunks(s, lambda r0: pltpu.sync_copy(
        acc.at[pl.ds(r0, RCH)], out_hbm.at[pl.ds(c * N + r0, RCH)]))


@functools.partial(
    pl.kernel,
    out_type=jax.ShapeDtypeStruct((NC * N, HALF), jnp.float32),
    mesh=_mesh,
    scratch_types=(
        [pltpu.VMEM((1, KS), jnp.int32)] * NBUF
        + [pltpu.VMEM((1, KS), jnp.int32)] * NBUF
        + [pltpu.VMEM((KS, HALF), jnp.float32)] * NBUF
        + [pltpu.VMEM_SHARED((N, HALF), jnp.float32)]
        + [pltpu.SemaphoreType.DMA] * (4 * NBUF)
    ),
)
def _sc_scatter(y_hbm, src3_hbm, dst3_hbm, out_hbm, *rest):
    srcb = rest[0:NBUF]
    dstb = rest[NBUF:2 * NBUF]
    rowsb = rest[2 * NBUF:3 * NBUF]
    acc = rest[3 * NBUF]
    isem = rest[3 * NBUF + 1:3 * NBUF + 1 + NBUF]
    gsem = rest[3 * NBUF + 1 + NBUF:3 * NBUF + 1 + 2 * NBUF]
    dsem = rest[3 * NBUF + 1 + 2 * NBUF:3 * NBUF + 1 + 3 * NBUF]
    ssem = rest[3 * NBUF + 1 + 3 * NBUF:3 * NBUF + 1 + 4 * NBUF]
    c = lax.axis_index("c")
    s = lax.axis_index("s")
    tid = c * NS + s

    def fetch_idx(j, b):
        pltpu.async_copy(src3_hbm.at[tid * NCHUNK + j], srcb[b], isem[b])

    def start_gather(j, b):
        pltpu.make_async_copy(src3_hbm.at[tid * NCHUNK + j], srcb[b],
                              isem[b]).wait()
        pltpu.async_copy(y_hbm.at[srcb[b].at[0]], rowsb[b], gsem[b])
        pltpu.async_copy(dst3_hbm.at[s * NCHUNK + j], dstb[b], dsem[b])

    def wait_gather(j, b):
        pltpu.make_async_copy(y_hbm.at[srcb[b].at[0]], rowsb[b], gsem[b]).wait()
        pltpu.make_async_copy(dst3_hbm.at[s * NCHUNK + j], dstb[b],
                              dsem[b]).wait()

    def wait_scat(b):
        pltpu.make_async_copy(rowsb[b], acc.at[dstb[b].at[0]], ssem[b]).wait()

    for k in range(NBUF):
        fetch_idx(k, k)
    start_gather(0, 0)
    start_gather(1, 1)
    _row_chunks(s, lambda r0: pltpu.sync_copy(
        y_hbm.at[pl.ds(c * N + r0, RCH)], acc.at[pl.ds(r0, RCH)]))
    plsc.subcore_barrier()

    NMAIN = (NCHUNK // NBUF - 1) * NBUF

    def block(g, carry):
        for b in range(NBUF):
            j = NBUF * g + b
            r = (b + 2) % NBUF
            wait_gather(j, b)
            fetch_idx(j + NBUF, b)
            pltpu.async_copy(rowsb[b], acc.at[dstb[b].at[0]], ssem[b], add=True)

            @pl.when(j >= 1)
            def _():
                wait_scat(r)
            start_gather(j + 2, r)
        return carry

    lax.fori_loop(0, NMAIN // NBUF, block, 0)
    for j in range(NMAIN, NCHUNK):
        b = j % NBUF
        wait_gather(j, b)
        if j + NBUF < NCHUNK:
            fetch_idx(j + NBUF, b)
        pltpu.sync_copy(rowsb[b], acc.at[dstb[b].at[0]], add=True)
        if j + 2 < NCHUNK:
            r = (j + 2) % NBUF
            if j - 1 < NMAIN:
                wait_scat(r)
            start_gather(j + 2, r)
    plsc.subcore_barrier()
    _row_chunks(s, lambda r0: pltpu.sync_copy(
        acc.at[pl.ds(r0, RCH)], out_hbm.at[pl.ds(c * N + r0, RCH)]))


BLK = 2000


def _dinv_block(degp_ref):
    d = degp_ref[0, :, 0:1] + degp_ref[1, :, 0:1] - 1.0
    return lax.rsqrt(d)


def _tc_xw_body(x_ref, w_ref, o_ref):
    o_ref[...] = jnp.dot(x_ref[...], w_ref[...],
                         preferred_element_type=jnp.float32)


def _tc_scale_body(xw_ref, degp_ref, y_ref):
    dinv = _dinv_block(degp_ref)
    y = xw_ref[...] * dinv
    y_ref[0] = y[:, :HALF]
    y_ref[1] = y[:, HALF:]


def _tc_mid_body(s_ref, degp_ref, b_ref, w_ref, y_ref):
    dinv = _dinv_block(degp_ref)
    h = jnp.concatenate([s_ref[0], s_ref[1]], axis=1) * dinv + b_ref[...]
    y = jnp.dot(h, w_ref[...], preferred_element_type=jnp.float32) * dinv
    y_ref[0] = y[:, :HALF]
    y_ref[1] = y[:, HALF:]


def _tc_final_body(s_ref, degp_ref, b_ref, o_ref):
    dinv = _dinv_block(degp_ref)
    o_ref[...] = jnp.concatenate([s_ref[0], s_ref[1]], axis=1) * dinv + b_ref[...]


_spec_x = pl.BlockSpec((BLK, D), lambda i: (i, 0))
_spec_degp = pl.BlockSpec((2, BLK, 16), lambda i: (0, i, 0))
_spec_w = pl.BlockSpec((D, D), lambda i: (0, 0))
_spec_b = pl.BlockSpec((1, D), lambda i: (0, 0))
_spec_y = pl.BlockSpec((2, BLK, HALF), lambda i: (0, i, 0))

_tc_xw = pl.pallas_call(
    _tc_xw_body,
    grid=(N // BLK,),
    in_specs=[_spec_x, _spec_w],
    out_specs=_spec_x,
    out_shape=jax.ShapeDtypeStruct((N, D), jnp.float32),
)

_tc_scale = pl.pallas_call(
    _tc_scale_body,
    grid=(N // BLK,),
    in_specs=[_spec_x, _spec_degp],
    out_specs=_spec_y,
    out_shape=jax.ShapeDtypeStruct((2, N, HALF), jnp.float32),
)

_tc_mid = pl.pallas_call(
    _tc_mid_body,
    grid=(N // BLK,),
    in_specs=[_spec_y, _spec_degp, _spec_b, _spec_w],
    out_specs=_spec_y,
    out_shape=jax.ShapeDtypeStruct((2, N, HALF), jnp.float32),
)

_tc_final = pl.pallas_call(
    _tc_final_body,
    grid=(N // BLK,),
    in_specs=[_spec_y, _spec_degp, _spec_b],
    out_specs=_spec_x,
    out_shape=jax.ShapeDtypeStruct((N, D), jnp.float32),
)


def kernel(x, edge_index, W1, b1, W2, b2, W3, b3):
    src = edge_index[0].astype(jnp.int32)
    dst = edge_index[1].astype(jnp.int32)
    src3 = jnp.concatenate([src, src + N]).reshape(NC * NS * NCHUNK, 1, KS)
    dst3 = dst.reshape(NS * NCHUNK, 1, KS)
    dstd3 = dst.reshape(NC * NS, NDCHUNK, KD)

    xw1 = _tc_xw(x, W1)
    degp = _sc_degree(dstd3).reshape(2, N, 16)
    y = _tc_scale(xw1, degp)
    for b, w in ((b1, W2), (b2, W3)):
        s = _sc_scatter(y.reshape(NC * N, HALF), src3, dst3).reshape(2, N, HALF)
        y = _tc_mid(s, degp, b.reshape(1, D), w)
    s = _sc_scatter(y.reshape(NC * N, HALF), src3, dst3).reshape(2, N, HALF)
    return _tc_final(s, degp, b3.reshape(1, D))
